# baseline (speedup 1.0000x reference)
import functools
import math

import jax
import jax.numpy as jnp
from jax import lax
from jax.experimental import pallas as pl
from jax.experimental.pallas import tpu as pltpu
from jax.experimental.pallas import tpu_sc as plsc

EMB = 128
SCALE = math.sqrt(EMB)
LANES = 16
CHUNK = 64
NBUF = 10


def _sc_embed(total, table, idx3d):
    info = plsc.get_sparse_core_info()
    nw = info.num_cores * info.num_subcores
    per_w = total // nw
    chunks = per_w // CHUNK
    assert chunks % NBUF == 0

    mesh = plsc.VectorSubcoreMesh(core_axis_name="c", subcore_axis_name="s")

    @functools.partial(
        pl.kernel,
        mesh=mesh,
        out_type=jax.ShapeDtypeStruct((total, EMB), jnp.float32),
        scratch_types=(
            [pltpu.VMEM((chunks, CHUNK), jnp.int32)]
            + [pltpu.VMEM((CHUNK, EMB), jnp.float32) for _ in range(NBUF)]
            + [pltpu.SemaphoreType.DMA, pltpu.SemaphoreType.DMA]
        ),
    )
    def k(table_hbm, idx_hbm, out_hbm, idx_v, *bufs_sems):
        bufs = list(bufs_sems[:NBUF])
        sem_in, sem_out = bufs_sems[NBUF:]
        wid = lax.axis_index("s") * info.num_cores + lax.axis_index("c")
        base = wid * per_w
        pltpu.sync_copy(idx_hbm.at[wid], idx_v)

        def gather(j, buf):
            return pltpu.make_async_copy(table_hbm.at[idx_v.at[j]], buf,
                                         sem_in)

        def out_copy(j, buf):
            return pltpu.make_async_copy(
                buf, out_hbm.at[pl.ds(base + j * CHUNK, CHUNK)], sem_out)

        def scale(buf):
            @plsc.parallel_loop(0, CHUNK, unroll=4)
            def row(r):
                for c in range(EMB // LANES):
                    sl = pl.ds(c * LANES, LANES)
                    buf[r, sl] = buf[r, sl] * SCALE

        for j in range(NBUF - 1):
            gather(j, bufs[j]).start()

        def outer(g, carry):
            for b in range(NBUF):
                j = g * NBUF + b
                gather(j, bufs[b]).wait()
                scale(bufs[b])

                @pl.when(j >= 1)
                def _():
                    out_copy(j - 1, bufs[(b - 1) % NBUF]).wait()

                out_copy(j, bufs[b]).start()

                @pl.when(j + NBUF - 1 < chunks)
                def _():
                    gather(j + NBUF - 1, bufs[(b + NBUF - 1) % NBUF]).start()

            return carry

        lax.fori_loop(0, chunks // NBUF, outer, 0)
        out_copy(chunks - 1, bufs[(chunks - 1) % NBUF]).wait()

    return k(table, idx3d)


def kernel(tokens, table):
    b, t = tokens.shape
    total = b * t
    info = plsc.get_sparse_core_info()
    nw = info.num_cores * info.num_subcores
    idx3d = tokens.astype(jnp.int32).T.reshape(nw, total // (nw * CHUNK), CHUNK)
    flat = _sc_embed(total, table, idx3d)
    return flat.reshape(t, b, EMB).transpose(1, 0, 2)


# speedup vs baseline: 1.0132x; 1.0132x over previous
import functools
import math

import jax
import jax.numpy as jnp
from jax import lax
from jax.experimental import pallas as pl
from jax.experimental.pallas import tpu as pltpu
from jax.experimental.pallas import tpu_sc as plsc

EMB = 128
SCALE = math.sqrt(EMB)
LANES = 16
CHUNK = 128
NBUF = 5


def _sc_embed(total, table, idx2d):
    info = plsc.get_sparse_core_info()
    nw = info.num_cores * info.num_subcores
    per_w = total // nw
    chunks = per_w // CHUNK
    assert chunks % NBUF == 0
    max_off = max((w * chunks) % 8 for w in range(nw))
    stage = chunks + max_off
    assert all((w * chunks) // 8 * 8 + stage <= total // CHUNK
               for w in range(nw))

    mesh = plsc.VectorSubcoreMesh(core_axis_name="c", subcore_axis_name="s")

    @functools.partial(
        pl.kernel,
        mesh=mesh,
        out_type=jax.ShapeDtypeStruct((total, EMB), jnp.float32),
        scratch_types=(
            [pltpu.VMEM((stage, CHUNK), jnp.int32)]
            + [pltpu.VMEM((CHUNK, EMB), jnp.float32) for _ in range(NBUF)]
            + [pltpu.SemaphoreType.DMA, pltpu.SemaphoreType.DMA]
        ),
    )
    def k(table_hbm, idx_hbm, out_hbm, idx_v, *bufs_sems):
        bufs = list(bufs_sems[:NBUF])
        sem_in, sem_out = bufs_sems[NBUF:]
        wid = lax.axis_index("s") * info.num_cores + lax.axis_index("c")
        base = wid * per_w
        start8 = (wid * chunks) // 8 * 8
        off0 = wid * chunks - start8
        pltpu.sync_copy(idx_hbm.at[pl.ds(start8, stage)], idx_v)

        def gather(j, buf):
            return pltpu.make_async_copy(table_hbm.at[idx_v.at[off0 + j]],
                                         buf, sem_in)

        def out_copy(j, buf):
            return pltpu.make_async_copy(
                buf, out_hbm.at[pl.ds(base + j * CHUNK, CHUNK)], sem_out)

        def scale(buf):
            @plsc.parallel_loop(0, CHUNK, unroll=4)
            def row(r):
                for c in range(EMB // LANES):
                    sl = pl.ds(c * LANES, LANES)
                    buf[r, sl] = buf[r, sl] * SCALE

        for j in range(NBUF - 1):
            gather(j, bufs[j]).start()

        def outer(g, carry):
            for b in range(NBUF):
                j = g * NBUF + b
                gather(j, bufs[b]).wait()
                scale(bufs[b])

                @pl.when(j >= 1)
                def _():
                    out_copy(j - 1, bufs[(b - 1) % NBUF]).wait()

                out_copy(j, bufs[b]).start()

                @pl.when(j + NBUF - 1 < chunks)
                def _():
                    gather(j + NBUF - 1, bufs[(b + NBUF - 1) % NBUF]).start()

            return carry

        lax.fori_loop(0, chunks // NBUF, outer, 0)
        out_copy(chunks - 1, bufs[(chunks - 1) % NBUF]).wait()

    return k(table, idx2d)


def kernel(tokens, table):
    b, t = tokens.shape
    total = b * t
    idx2d = tokens.astype(jnp.int32).T.reshape(total // CHUNK, CHUNK)
    flat = _sc_embed(total, table, idx2d)
    return flat.reshape(t, b, EMB).transpose(1, 0, 2)


# speedup vs baseline: 1.0154x; 1.0022x over previous
import functools
import math

import jax
import jax.numpy as jnp
from jax import lax
from jax.experimental import pallas as pl
from jax.experimental.pallas import tpu as pltpu
from jax.experimental.pallas import tpu_sc as plsc

EMB = 128
SCALE = math.sqrt(EMB)
LANES = 16
CHUNK = 128
NBUF = 5


def _sc_embed(total, table, idx3d):
    info = plsc.get_sparse_core_info()
    nw = info.num_cores * info.num_subcores
    per_w = total // nw
    chunks = per_w // CHUNK
    assert chunks % NBUF == 0

    mesh = plsc.VectorSubcoreMesh(core_axis_name="c", subcore_axis_name="s")

    @functools.partial(
        pl.kernel,
        mesh=mesh,
        out_type=jax.ShapeDtypeStruct((total, EMB), jnp.float32),
        scratch_types=(
            [pltpu.VMEM((chunks, CHUNK), jnp.int32)]
            + [pltpu.VMEM((CHUNK, EMB), jnp.float32) for _ in range(NBUF)]
            + [pltpu.SemaphoreType.DMA, pltpu.SemaphoreType.DMA]
        ),
    )
    def k(table_hbm, idx_hbm, out_hbm, idx_v, *bufs_sems):
        bufs = list(bufs_sems[:NBUF])
        sem_in, sem_out = bufs_sems[NBUF:]
        wid = lax.axis_index("s") * info.num_cores + lax.axis_index("c")
        base = wid * per_w
        pltpu.sync_copy(idx_hbm.at[wid], idx_v)

        def gather(j, buf):
            return pltpu.make_async_copy(table_hbm.at[idx_v.at[j]], buf,
                                         sem_in)

        def out_copy(j, buf):
            return pltpu.make_async_copy(
                buf, out_hbm.at[pl.ds(base + j * CHUNK, CHUNK)], sem_out)

        def scale(buf):
            @plsc.parallel_loop(0, CHUNK, unroll=4)
            def row(r):
                for c in range(EMB // LANES):
                    sl = pl.ds(c * LANES, LANES)
                    buf[r, sl] = buf[r, sl] * SCALE

        for j in range(NBUF - 2):
            gather(j, bufs[j]).start()

        def outer(g, carry):
            for b in range(NBUF):
                j = g * NBUF + b
                gather(j, bufs[b]).wait()
                scale(bufs[b])

                @pl.when(j >= 2)
                def _():
                    out_copy(j - 2, bufs[(b - 2) % NBUF]).wait()

                out_copy(j, bufs[b]).start()

                @pl.when(j + NBUF - 2 < chunks)
                def _():
                    gather(j + NBUF - 2, bufs[(b + NBUF - 2) % NBUF]).start()

            return carry

        lax.fori_loop(0, chunks // NBUF, outer, 0)
        out_copy(chunks - 2, bufs[(chunks - 2) % NBUF]).wait()
        out_copy(chunks - 1, bufs[(chunks - 1) % NBUF]).wait()

    return k(table, idx3d)


def kernel(tokens, table):
    b, t = tokens.shape
    total = b * t
    info = plsc.get_sparse_core_info()
    nw = info.num_cores * info.num_subcores
    idx3d = tokens.astype(jnp.int32).T.reshape(nw, total // (nw * CHUNK), CHUNK)
    flat = _sc_embed(total, table, idx3d)
    return flat.reshape(t, b, EMB).transpose(1, 0, 2)


# speedup vs baseline: 1.0177x; 1.0023x over previous
import functools
import math

import jax
import jax.numpy as jnp
from jax import lax
from jax.experimental import pallas as pl
from jax.experimental.pallas import tpu as pltpu
from jax.experimental.pallas import tpu_sc as plsc

EMB = 128
SCALE = math.sqrt(EMB)
LANES = 16
CHUNK = 128
NBUF = 5


def _sc_embed(total, table, idx3d):
    info = plsc.get_sparse_core_info()
    nw = info.num_cores * info.num_subcores
    per_w = total // nw
    chunks = per_w // CHUNK
    assert chunks % NBUF == 0

    mesh = plsc.VectorSubcoreMesh(core_axis_name="c", subcore_axis_name="s")

    @functools.partial(
        pl.kernel,
        mesh=mesh,
        out_type=jax.ShapeDtypeStruct((total, EMB), jnp.float32),
        scratch_types=(
            [pltpu.VMEM((chunks, CHUNK), jnp.int32)]
            + [pltpu.VMEM((CHUNK, EMB), jnp.float32) for _ in range(NBUF)]
            + [pltpu.SemaphoreType.DMA, pltpu.SemaphoreType.DMA]
        ),
    )
    def k(table_hbm, idx_hbm, out_hbm, idx_v, *bufs_sems):
        bufs = list(bufs_sems[:NBUF])
        sem_in, sem_out = bufs_sems[NBUF:]
        wid = lax.axis_index("s") * info.num_cores + lax.axis_index("c")
        base = wid * per_w
        pltpu.sync_copy(idx_hbm.at[wid], idx_v)

        def gather(j, buf):
            return pltpu.make_async_copy(table_hbm.at[idx_v.at[j]], buf,
                                         sem_in)

        def out_copy(j, buf):
            return pltpu.make_async_copy(
                buf, out_hbm.at[pl.ds(base + j * CHUNK, CHUNK)], sem_out)

        def scale(buf):
            @plsc.parallel_loop(0, CHUNK, unroll=4)
            def row(r):
                for c in range(EMB // LANES):
                    sl = pl.ds(c * LANES, LANES)
                    buf[r, sl] = buf[r, sl] * SCALE

        for j in range(NBUF - 1):
            gather(j, bufs[j]).start()

        def outer(g, carry):
            for b in range(NBUF):
                j = g * NBUF + b
                gather(j, bufs[b]).wait()
                scale(bufs[b])

                @pl.when(j >= 1)
                def _():
                    out_copy(j - 1, bufs[(b - 1) % NBUF]).wait()

                out_copy(j, bufs[b]).start()

                @pl.when(j + NBUF - 1 < chunks)
                def _():
                    gather(j + NBUF - 1, bufs[(b + NBUF - 1) % NBUF]).start()

            return carry

        lax.fori_loop(0, chunks // NBUF, outer, 0)
        out_copy(chunks - 1, bufs[(chunks - 1) % NBUF]).wait()

    return k(table, idx3d)


def kernel(tokens, table):
    b, t = tokens.shape
    total = b * t
    info = plsc.get_sparse_core_info()
    nw = info.num_cores * info.num_subcores
    idx3d = tokens.astype(jnp.int32).T.reshape(nw, total // (nw * CHUNK), CHUNK)
    flat = _sc_embed(total, table, idx3d)
    return flat.reshape(t, b, EMB).transpose(1, 0, 2)
